# baseline (speedup 1.0000x reference)
import functools

import jax
import jax.numpy as jnp
from jax.experimental import pallas as pl
from jax.experimental.pallas import tpu as pltpu

_NUM_EMBEDDINGS = 1024
_EMBEDDING_DIM = 64
_COMMITMENT_COST = 0.25
_LB = 2048
_KAUG = 72


def _vq_kernel(x_ref, wa_ref, wt_ref, out_ref, loss_ref, perp_ref,
               counts_ref, sse_ref, *, n_rows, n_elems):
    b = pl.program_id(0)
    l = pl.program_id(1)
    nb = pl.num_programs(0)
    nl = pl.num_programs(1)

    @pl.when((b == 0) & (l == 0))
    def _init():
        counts_ref[...] = jnp.zeros_like(counts_ref)
        sse_ref[0] = 0.0

    xa = x_ref[...]
    d = jnp.dot(wa_ref[...], xa, preferred_element_type=jnp.float32)
    m = jnp.min(d, axis=0, keepdims=True)
    iota = jax.lax.broadcasted_iota(jnp.int32, d.shape, 0)
    idx = jnp.min(jnp.where(d <= m, iota, _NUM_EMBEDDINGS), axis=0,
                  keepdims=True)
    onehot = (iota == idx).astype(jnp.float32)
    out_ref[...] = jnp.dot(wt_ref[...], onehot,
                           preferred_element_type=jnp.float32)
    sse_ref[0] += jnp.sum(m) + (jnp.sum(xa * xa) - float(_LB))
    counts_ref[...] += jnp.sum(onehot, axis=1, keepdims=True)

    @pl.when((b == nb - 1) & (l == nl - 1))
    def _finalize():
        loss = (1.0 + _COMMITMENT_COST) * sse_ref[0] / n_elems
        loss_ref[...] = jnp.reshape(loss, (1, 1))
        p = counts_ref[...] / n_rows
        perp = jnp.exp(-jnp.sum(p * jnp.log(p + 1e-10)))
        perp_ref[...] = jnp.reshape(perp, (1, 1))


def kernel(inputs, weight):
    batch, c, length = inputs.shape
    n_rows = batch * length
    n_elems = batch * length * c

    last = jnp.transpose(inputs[-1, :, length - _NUM_EMBEDDINGS:], (1, 0))
    w = jnp.where(jnp.all(weight == 0.0), last, weight)

    xa = jnp.concatenate(
        [inputs,
         jnp.ones((batch, 1, length), jnp.float32),
         jnp.zeros((batch, _KAUG - c - 1, length), jnp.float32)],
        axis=1).reshape(batch * _KAUG, length)
    wa = jnp.concatenate(
        [-2.0 * w,
         jnp.sum(w * w, axis=1, keepdims=True),
         jnp.zeros((_NUM_EMBEDDINGS, _KAUG - c - 1), jnp.float32)],
        axis=1)

    grid = (batch, length // _LB)
    body = functools.partial(_vq_kernel, n_rows=float(n_rows),
                             n_elems=float(n_elems))
    q, loss, perp = pl.pallas_call(
        body,
        grid=grid,
        in_specs=[
            pl.BlockSpec((_KAUG, _LB), lambda b, l: (b, l)),
            pl.BlockSpec((_NUM_EMBEDDINGS, _KAUG), lambda b, l: (0, 0)),
            pl.BlockSpec((_EMBEDDING_DIM, _NUM_EMBEDDINGS), lambda b, l: (0, 0)),
        ],
        out_specs=[
            pl.BlockSpec((c, _LB), lambda b, l: (b, l)),
            pl.BlockSpec((1, 1), lambda b, l: (0, 0)),
            pl.BlockSpec((1, 1), lambda b, l: (0, 0)),
        ],
        out_shape=[
            jax.ShapeDtypeStruct((batch * c, length), jnp.float32),
            jax.ShapeDtypeStruct((1, 1), jnp.float32),
            jax.ShapeDtypeStruct((1, 1), jnp.float32),
        ],
        scratch_shapes=[
            pltpu.VMEM((_NUM_EMBEDDINGS, 1), jnp.float32),
            pltpu.SMEM((1,), jnp.float32),
        ],
        compiler_params=pltpu.CompilerParams(
            dimension_semantics=("arbitrary", "arbitrary")),
    )(xa, wa, w.T)
    return (loss[0, 0], q.reshape(batch, c, length), perp[0, 0])


# speedup vs baseline: 1.0533x; 1.0533x over previous
import functools

import jax
import jax.numpy as jnp
from jax.experimental import pallas as pl
from jax.experimental.pallas import tpu as pltpu

_NUM_EMBEDDINGS = 1024
_EMBEDDING_DIM = 64
_COMMITMENT_COST = 0.25
_LB = 2048


def _vq_kernel(x_ref, w2_ref, wsq_ref, wt_ref, out_ref, loss_ref, perp_ref,
               counts_ref, sse_ref, *, n_rows, n_elems):
    b = pl.program_id(0)
    l = pl.program_id(1)
    nb = pl.num_programs(0)
    nl = pl.num_programs(1)

    @pl.when((b == 0) & (l == 0))
    def _init():
        counts_ref[...] = jnp.zeros_like(counts_ref)
        sse_ref[0] = 0.0

    x = x_ref[...]
    d = jnp.dot(w2_ref[...], x, preferred_element_type=jnp.float32) + wsq_ref[...]
    m = jnp.min(d, axis=0, keepdims=True)
    oh = (d <= m).astype(jnp.float32)
    cnt = jnp.sum(oh, axis=1, keepdims=True)
    matches = jnp.sum(cnt)
    sse_ref[0] += jnp.sum(m) + jnp.sum(x * x)

    @pl.when(matches <= float(_LB))
    def _fast():
        out_ref[...] = jnp.dot(wt_ref[...], oh,
                               preferred_element_type=jnp.float32)
        counts_ref[...] += cnt

    @pl.when(matches > float(_LB))
    def _ties():
        iota = jax.lax.broadcasted_iota(jnp.int32, d.shape, 0)
        idx = jnp.min(jnp.where(d <= m, iota, _NUM_EMBEDDINGS), axis=0,
                      keepdims=True)
        onehot = (iota == idx).astype(jnp.float32)
        out_ref[...] = jnp.dot(wt_ref[...], onehot,
                               preferred_element_type=jnp.float32)
        counts_ref[...] += jnp.sum(onehot, axis=1, keepdims=True)

    @pl.when((b == nb - 1) & (l == nl - 1))
    def _finalize():
        loss = (1.0 + _COMMITMENT_COST) * sse_ref[0] / n_elems
        loss_ref[...] = jnp.reshape(loss, (1, 1))
        p = counts_ref[...] / n_rows
        perp = jnp.exp(-jnp.sum(p * jnp.log(p + 1e-10)))
        perp_ref[...] = jnp.reshape(perp, (1, 1))


def kernel(inputs, weight):
    batch, c, length = inputs.shape
    n_rows = batch * length
    n_elems = batch * length * c

    last = jnp.transpose(inputs[-1, :, length - _NUM_EMBEDDINGS:], (1, 0))
    w = jnp.where(jnp.all(weight == 0.0), last, weight)

    x2d = inputs.reshape(batch * c, length)
    grid = (batch, length // _LB)
    body = functools.partial(_vq_kernel, n_rows=float(n_rows),
                             n_elems=float(n_elems))
    q, loss, perp = pl.pallas_call(
        body,
        grid=grid,
        in_specs=[
            pl.BlockSpec((c, _LB), lambda b, l: (b, l)),
            pl.BlockSpec((_NUM_EMBEDDINGS, _EMBEDDING_DIM), lambda b, l: (0, 0)),
            pl.BlockSpec((_NUM_EMBEDDINGS, 1), lambda b, l: (0, 0)),
            pl.BlockSpec((_EMBEDDING_DIM, _NUM_EMBEDDINGS), lambda b, l: (0, 0)),
        ],
        out_specs=[
            pl.BlockSpec((c, _LB), lambda b, l: (b, l)),
            pl.BlockSpec((1, 1), lambda b, l: (0, 0)),
            pl.BlockSpec((1, 1), lambda b, l: (0, 0)),
        ],
        out_shape=[
            jax.ShapeDtypeStruct((batch * c, length), jnp.float32),
            jax.ShapeDtypeStruct((1, 1), jnp.float32),
            jax.ShapeDtypeStruct((1, 1), jnp.float32),
        ],
        scratch_shapes=[
            pltpu.VMEM((_NUM_EMBEDDINGS, 1), jnp.float32),
            pltpu.SMEM((1,), jnp.float32),
        ],
        compiler_params=pltpu.CompilerParams(
            dimension_semantics=("arbitrary", "arbitrary")),
    )(x2d, -2.0 * w, jnp.sum(w * w, axis=1, keepdims=True), w.T)
    return (loss[0, 0], q.reshape(batch, c, length), perp[0, 0])


# speedup vs baseline: 1.2515x; 1.1881x over previous
import functools

import jax
import jax.numpy as jnp
from jax.experimental import pallas as pl
from jax.experimental.pallas import tpu as pltpu

_NUM_EMBEDDINGS = 1024
_EMBEDDING_DIM = 64
_COMMITMENT_COST = 0.25
_LB = 4096


def _vq_kernel(x_ref, w2_ref, wsq_ref, wt_ref, out_ref, loss_ref, perp_ref,
               counts_ref, sse_ref, *, n_rows, n_elems):
    b = pl.program_id(0)
    l = pl.program_id(1)
    nb = pl.num_programs(0)
    nl = pl.num_programs(1)

    @pl.when((b == 0) & (l == 0))
    def _init():
        counts_ref[...] = jnp.zeros_like(counts_ref)
        sse_ref[0] = 0.0

    x = x_ref[...]
    d = jnp.dot(w2_ref[...], x, preferred_element_type=jnp.float32) + wsq_ref[...]
    m = jnp.min(d, axis=0, keepdims=True)
    idx = jnp.argmin(d, axis=0)[None, :]
    iota = jax.lax.broadcasted_iota(jnp.int32, d.shape, 0)
    onehot = (iota == idx).astype(jnp.float32)
    out_ref[...] = jnp.dot(wt_ref[...], onehot,
                           preferred_element_type=jnp.float32)
    sse_ref[0] += jnp.sum(m) + jnp.sum(x * x)
    counts_ref[...] += jnp.sum(onehot, axis=1, keepdims=True)

    @pl.when((b == nb - 1) & (l == nl - 1))
    def _finalize():
        loss = (1.0 + _COMMITMENT_COST) * sse_ref[0] / n_elems
        loss_ref[...] = jnp.reshape(loss, (1, 1))
        p = counts_ref[...] / n_rows
        perp = jnp.exp(-jnp.sum(p * jnp.log(p + 1e-10)))
        perp_ref[...] = jnp.reshape(perp, (1, 1))


def kernel(inputs, weight):
    batch, c, length = inputs.shape
    n_rows = batch * length
    n_elems = batch * length * c

    last = jnp.transpose(inputs[-1, :, length - _NUM_EMBEDDINGS:], (1, 0))
    w = jnp.where(jnp.all(weight == 0.0), last, weight)

    x2d = inputs.reshape(batch * c, length)
    grid = (batch, length // _LB)
    body = functools.partial(_vq_kernel, n_rows=float(n_rows),
                             n_elems=float(n_elems))
    q, loss, perp = pl.pallas_call(
        body,
        grid=grid,
        in_specs=[
            pl.BlockSpec((c, _LB), lambda b, l: (b, l)),
            pl.BlockSpec((_NUM_EMBEDDINGS, _EMBEDDING_DIM), lambda b, l: (0, 0)),
            pl.BlockSpec((_NUM_EMBEDDINGS, 1), lambda b, l: (0, 0)),
            pl.BlockSpec((_EMBEDDING_DIM, _NUM_EMBEDDINGS), lambda b, l: (0, 0)),
        ],
        out_specs=[
            pl.BlockSpec((c, _LB), lambda b, l: (b, l)),
            pl.BlockSpec((1, 1), lambda b, l: (0, 0)),
            pl.BlockSpec((1, 1), lambda b, l: (0, 0)),
        ],
        out_shape=[
            jax.ShapeDtypeStruct((batch * c, length), jnp.float32),
            jax.ShapeDtypeStruct((1, 1), jnp.float32),
            jax.ShapeDtypeStruct((1, 1), jnp.float32),
        ],
        scratch_shapes=[
            pltpu.VMEM((_NUM_EMBEDDINGS, 1), jnp.float32),
            pltpu.SMEM((1,), jnp.float32),
        ],
        compiler_params=pltpu.CompilerParams(
            dimension_semantics=("arbitrary", "arbitrary")),
    )(x2d, -2.0 * w, jnp.sum(w * w, axis=1, keepdims=True), w.T)
    return (loss[0, 0], q.reshape(batch, c, length), perp[0, 0])


# speedup vs baseline: 1.4190x; 1.1338x over previous
import functools

import jax
import jax.numpy as jnp
from jax.experimental import pallas as pl
from jax.experimental.pallas import tpu as pltpu

_NUM_EMBEDDINGS = 1024
_EMBEDDING_DIM = 64
_COMMITMENT_COST = 0.25
_LB = 4096


def _vq_kernel(x_ref, w2_ref, wsq_ref, wt_ref, out_ref, loss_ref, perp_ref,
               counts_ref, sse_ref, *, n_rows, n_elems):
    b = pl.program_id(0)
    l = pl.program_id(1)
    nb = pl.num_programs(0)
    nl = pl.num_programs(1)

    @pl.when((b == 0) & (l == 0))
    def _init():
        counts_ref[...] = jnp.zeros_like(counts_ref)
        sse_ref[0] = 0.0

    x = x_ref[...]
    d = jnp.dot(w2_ref[...], x, preferred_element_type=jnp.float32) + wsq_ref[...]
    idx = jnp.argmin(d, axis=0)[None, :]
    iota = jax.lax.broadcasted_iota(jnp.int32, d.shape, 0)
    onehot = (iota == idx).astype(jnp.float32)
    q = jnp.dot(wt_ref[...], onehot, preferred_element_type=jnp.float32)
    out_ref[...] = q
    sse_ref[0] += jnp.sum((q - x) * (q - x))
    counts_ref[...] += jnp.sum(onehot, axis=1, keepdims=True)

    @pl.when((b == nb - 1) & (l == nl - 1))
    def _finalize():
        loss = (1.0 + _COMMITMENT_COST) * sse_ref[0] / n_elems
        loss_ref[...] = jnp.reshape(loss, (1, 1))
        p = counts_ref[...] / n_rows
        perp = jnp.exp(-jnp.sum(p * jnp.log(p + 1e-10)))
        perp_ref[...] = jnp.reshape(perp, (1, 1))


def kernel(inputs, weight):
    batch, c, length = inputs.shape
    n_rows = batch * length
    n_elems = batch * length * c

    last = jnp.transpose(inputs[-1, :, length - _NUM_EMBEDDINGS:], (1, 0))
    w = jnp.where(jnp.all(weight == 0.0), last, weight)

    x2d = inputs.reshape(batch * c, length)
    grid = (batch, length // _LB)
    body = functools.partial(_vq_kernel, n_rows=float(n_rows),
                             n_elems=float(n_elems))
    q, loss, perp = pl.pallas_call(
        body,
        grid=grid,
        in_specs=[
            pl.BlockSpec((c, _LB), lambda b, l: (b, l)),
            pl.BlockSpec((_NUM_EMBEDDINGS, _EMBEDDING_DIM), lambda b, l: (0, 0)),
            pl.BlockSpec((_NUM_EMBEDDINGS, 1), lambda b, l: (0, 0)),
            pl.BlockSpec((_EMBEDDING_DIM, _NUM_EMBEDDINGS), lambda b, l: (0, 0)),
        ],
        out_specs=[
            pl.BlockSpec((c, _LB), lambda b, l: (b, l)),
            pl.BlockSpec((1, 1), lambda b, l: (0, 0)),
            pl.BlockSpec((1, 1), lambda b, l: (0, 0)),
        ],
        out_shape=[
            jax.ShapeDtypeStruct((batch * c, length), jnp.float32),
            jax.ShapeDtypeStruct((1, 1), jnp.float32),
            jax.ShapeDtypeStruct((1, 1), jnp.float32),
        ],
        scratch_shapes=[
            pltpu.VMEM((_NUM_EMBEDDINGS, 1), jnp.float32),
            pltpu.SMEM((1,), jnp.float32),
        ],
        compiler_params=pltpu.CompilerParams(
            dimension_semantics=("arbitrary", "arbitrary")),
    )(x2d, -2.0 * w, jnp.sum(w * w, axis=1, keepdims=True), w.T)
    return (loss[0, 0], q.reshape(batch, c, length), perp[0, 0])
